# baseline (speedup 1.0000x reference)
import functools

import jax
import jax.numpy as jnp
from jax import lax
from jax.experimental import pallas as pl
from jax.experimental.pallas import tpu as pltpu
from jax.experimental.pallas import tpu_sc as plsc

_HIDDEN = 768
_EPS = 1e-5
_LANES = 16
_NV = _HIDDEN // _LANES

_NC, _NS = 2, 16
_NW = _NC * _NS
_TOKENS = 4 * 4096
_TW = _TOKENS // _NW
_R = 32
_NCHUNK = _TW // _R
_NBUF = 4

_GATHER_DNUMS = lax.GatherDimensionNumbers(
    offset_dims=(), collapsed_slice_dims=(0,), start_index_map=(0,)
)


def _permute16(v, idx):
    return lax.gather(
        v,
        idx[:, None],
        _GATHER_DNUMS,
        slice_sizes=(1,),
        mode=lax.GatherScatterMode.PROMISE_IN_BOUNDS,
    )


def _allreduce_sum16(v):
    idx = lax.iota(jnp.int32, 16)
    for off in (8, 4, 2, 1):
        v = v + _permute16(v, idx ^ off)
    return v


def _rsqrt16(x):
    i = lax.bitcast_convert_type(x, jnp.int32)
    i = jnp.int32(0x5F3759DF) - lax.shift_right_logical(i, 1)
    y = lax.bitcast_convert_type(i, jnp.float32)
    for _ in range(3):
        y = y * (1.5 - 0.5 * x * y * y)
    return y


def _layernorm_chunk(rows_v):
    @plsc.parallel_loop(0, _R, unroll=2)
    def row_body(r):
        acc = jnp.zeros((_LANES,), jnp.float32)
        acc2 = jnp.zeros((_LANES,), jnp.float32)
        vals = []
        for j in range(_NV):
            v = rows_v[r, pl.ds(j * _LANES, _LANES)]
            vals.append(v)
            acc = acc + v
            acc2 = acc2 + v * v
        mean_v = _allreduce_sum16(acc) * (1.0 / _HIDDEN)
        var_v = _allreduce_sum16(acc2) * (1.0 / _HIDDEN) - mean_v * mean_v
        rinv_v = _rsqrt16(var_v + _EPS)
        for j in range(_NV):
            rows_v[r, pl.ds(j * _LANES, _LANES)] = (vals[j] - mean_v) * rinv_v


def _body(
    ids_hbm, table_hbm, out_hbm, idx_v,
    rows0, rows1, rows2, rows3, g0, g1, g2, g3, o0, o1, o2, o3,
):
    wid = lax.axis_index("s") * _NC + lax.axis_index("c")
    base = wid * _TW
    pltpu.sync_copy(ids_hbm.at[pl.ds(base, _TW)], idx_v)

    bufs = (rows0, rows1, rows2, rows3)
    gsems = (g0, g1, g2, g3)
    osems = (o0, o1, o2, o3)

    def gather(c, k):
        return pltpu.make_async_copy(
            table_hbm.at[idx_v.at[pl.ds(c * _R, _R)]], bufs[k], gsems[k]
        )

    def writeback(c, k):
        return pltpu.make_async_copy(
            bufs[k], out_hbm.at[pl.ds(base + c * _R, _R)], osems[k]
        )

    nsuper = _NCHUNK // _NBUF
    gather(0, 0).start()
    gather(1, 1).start()

    def superstep(s, carry):
        c0 = _NBUF * s
        for k in range(_NBUF):
            c = c0 + k
            kn = (k + 2) % _NBUF
            gather(c, k).wait()
            _layernorm_chunk(bufs[k])
            writeback(c, k).start()
            if k < 2:
                @pl.when(s > 0)
                def _():
                    writeback(c - 2, kn).wait()

                gather(c + 2, kn).start()
            else:
                writeback(c - 2, kn).wait()

                @pl.when(s < nsuper - 1)
                def _():
                    gather(c + 2, kn).start()

        return carry

    lax.fori_loop(0, nsuper, superstep, 0)
    writeback(_NCHUNK - 2, (_NCHUNK - 2) % _NBUF).wait()
    writeback(_NCHUNK - 1, (_NCHUNK - 1) % _NBUF).wait()


_mesh = plsc.VectorSubcoreMesh(
    core_axis_name="c", subcore_axis_name="s", num_cores=_NC, num_subcores=_NS
)

_embed_ln = functools.partial(
    pl.kernel,
    out_type=jax.ShapeDtypeStruct((_TOKENS, _HIDDEN), jnp.float32),
    mesh=_mesh,
    scratch_types=[
        pltpu.VMEM((_TW,), jnp.int32),
        pltpu.VMEM((_R, _HIDDEN), jnp.float32),
        pltpu.VMEM((_R, _HIDDEN), jnp.float32),
        pltpu.VMEM((_R, _HIDDEN), jnp.float32),
        pltpu.VMEM((_R, _HIDDEN), jnp.float32),
        pltpu.SemaphoreType.DMA,
        pltpu.SemaphoreType.DMA,
        pltpu.SemaphoreType.DMA,
        pltpu.SemaphoreType.DMA,
        pltpu.SemaphoreType.DMA,
        pltpu.SemaphoreType.DMA,
        pltpu.SemaphoreType.DMA,
        pltpu.SemaphoreType.DMA,
    ],
)(_body)


@jax.jit
def kernel(input_ids, tok_embeddings):
    b, s = input_ids.shape
    ids = input_ids.reshape(-1).astype(jnp.int32)
    out = _embed_ln(ids, tok_embeddings)
    return out.reshape(b, s, _HIDDEN)


# speedup vs baseline: 1.1767x; 1.1767x over previous
import functools

import jax
import jax.numpy as jnp
from jax import lax
from jax.experimental import pallas as pl
from jax.experimental.pallas import tpu as pltpu
from jax.experimental.pallas import tpu_sc as plsc

_HIDDEN = 768
_EPS = 1e-5
_LANES = 16
_NV = _HIDDEN // _LANES

_NC, _NS = 2, 16
_NW = _NC * _NS
_TOKENS = 4 * 4096
_TW = _TOKENS // _NW
_R = 32
_NCHUNK = _TW // _R
_NBUF = 4

_GATHER_DNUMS = lax.GatherDimensionNumbers(
    offset_dims=(), collapsed_slice_dims=(0,), start_index_map=(0,)
)


def _permute16(v, idx):
    return lax.gather(
        v,
        idx[:, None],
        _GATHER_DNUMS,
        slice_sizes=(1,),
        mode=lax.GatherScatterMode.PROMISE_IN_BOUNDS,
    )


def _allreduce_sum16(v):
    idx = lax.iota(jnp.int32, 16)
    for off in (8, 4, 2, 1):
        v = v + _permute16(v, idx ^ off)
    return v


def _rsqrt16(x):
    i = lax.bitcast_convert_type(x, jnp.int32)
    i = jnp.int32(0x5F3759DF) - lax.shift_right_logical(i, 1)
    y = lax.bitcast_convert_type(i, jnp.float32)
    for _ in range(3):
        y = y * (1.5 - 0.5 * x * y * y)
    return y


def _layernorm_chunk(rows_v):
    @plsc.parallel_loop(0, _R, unroll=1)
    def row_body(r):
        acc = jnp.zeros((_LANES,), jnp.float32)
        acc2 = jnp.zeros((_LANES,), jnp.float32)
        vals = []
        for j in range(_NV):
            v = rows_v[r, pl.ds(j * _LANES, _LANES)]
            vals.append(v)
            acc = acc + v
            acc2 = acc2 + v * v
        mean_v = _allreduce_sum16(acc) * (1.0 / _HIDDEN)
        var_v = _allreduce_sum16(acc2) * (1.0 / _HIDDEN) - mean_v * mean_v
        rinv_v = _rsqrt16(var_v + _EPS)
        for j in range(_NV):
            rows_v[r, pl.ds(j * _LANES, _LANES)] = (vals[j] - mean_v) * rinv_v


def _body(
    ids_hbm, table_hbm, out_hbm, idx_v,
    rows0, rows1, rows2, rows3, g0, g1, g2, g3, o0, o1, o2, o3,
):
    wid = lax.axis_index("s") * _NC + lax.axis_index("c")
    base = wid * _TW
    pltpu.sync_copy(ids_hbm.at[pl.ds(base, _TW)], idx_v)

    bufs = (rows0, rows1, rows2, rows3)
    gsems = (g0, g1, g2, g3)
    osems = (o0, o1, o2, o3)

    def gather(c, k):
        return pltpu.make_async_copy(
            table_hbm.at[idx_v.at[pl.ds(c * _R, _R)]], bufs[k], gsems[k]
        )

    def writeback(c, k):
        return pltpu.make_async_copy(
            bufs[k], out_hbm.at[pl.ds(base + c * _R, _R)], osems[k]
        )

    nsuper = _NCHUNK // _NBUF
    gather(0, 0).start()
    gather(1, 1).start()

    def superstep(s, carry):
        c0 = _NBUF * s
        for k in range(_NBUF):
            c = c0 + k
            kn = (k + 2) % _NBUF
            gather(c, k).wait()
            _layernorm_chunk(bufs[k])
            writeback(c, k).start()
            if k < 2:
                @pl.when(s > 0)
                def _():
                    writeback(c - 2, kn).wait()

                gather(c + 2, kn).start()
            else:
                writeback(c - 2, kn).wait()

                @pl.when(s < nsuper - 1)
                def _():
                    gather(c + 2, kn).start()

        return carry

    lax.fori_loop(0, nsuper, superstep, 0)
    writeback(_NCHUNK - 2, (_NCHUNK - 2) % _NBUF).wait()
    writeback(_NCHUNK - 1, (_NCHUNK - 1) % _NBUF).wait()


_mesh = plsc.VectorSubcoreMesh(
    core_axis_name="c", subcore_axis_name="s", num_cores=_NC, num_subcores=_NS
)

_embed_ln = functools.partial(
    pl.kernel,
    out_type=jax.ShapeDtypeStruct((_TOKENS, _HIDDEN), jnp.float32),
    mesh=_mesh,
    scratch_types=[
        pltpu.VMEM((_TW,), jnp.int32),
        pltpu.VMEM((_R, _HIDDEN), jnp.float32),
        pltpu.VMEM((_R, _HIDDEN), jnp.float32),
        pltpu.VMEM((_R, _HIDDEN), jnp.float32),
        pltpu.VMEM((_R, _HIDDEN), jnp.float32),
        pltpu.SemaphoreType.DMA,
        pltpu.SemaphoreType.DMA,
        pltpu.SemaphoreType.DMA,
        pltpu.SemaphoreType.DMA,
        pltpu.SemaphoreType.DMA,
        pltpu.SemaphoreType.DMA,
        pltpu.SemaphoreType.DMA,
        pltpu.SemaphoreType.DMA,
    ],
)(_body)


@jax.jit
def kernel(input_ids, tok_embeddings):
    b, s = input_ids.shape
    ids = input_ids.reshape(-1).astype(jnp.int32)
    out = _embed_ln(ids, tok_embeddings)
    return out.reshape(b, s, _HIDDEN)
